# baseline (speedup 1.0000x reference)
import functools

import jax
import jax.numpy as jnp
from jax import lax
from jax.experimental import pallas as pl
from jax.experimental.pallas import tpu as pltpu
from jax.experimental.pallas import tpu_sc as plsc

_N = 16000
_D = 128
_E = 40000

_NC = 2
_NS = 16
_HALF = _N // _NC
_L = 16

_CH = 64
_NCH = 40
_EPAD = _NS * _NCH * _CH
_NPAD = 8064
_ZROWS = _NPAD // _NS
_CROWS = 1000

_TYPES = ["router", "pe", "task", "start_task", "end_task", "link"]
_RELS = [("router", "router"), ("router", "pe"), ("pe", "router"), ("task", "pe"),
         ("start_task", "task"), ("task", "task"), ("task", "end_task"),
         ("task", "link"), ("start_task", "link"), ("link", "router"),
         ("link", "task"), ("link", "end_task"), ("end_task", "pe"),
         ("start_task", "pe")]


_DEPTH = 6
_GROUPS = [list(range(g, min(g + _DEPTH, _NCH))) for g in range(0, _NCH, _DEPTH)]


def _sc_agg_body(x_hbm, src_hbm, dst_hbm, z_hbm, out_hbm,
                 sidx_all, didx_all, rows_v,
                 lidx0, lidx1, lidx2, lidx3, lidx4, lidx5,
                 agg_sh, gsem, ssem):
    lidx = (lidx0, lidx1, lidx2, lidx3, lidx4, lidx5)
    c = lax.axis_index("c")
    s = lax.axis_index("s")
    base = c * _HALF
    trash = _HALF + s
    pltpu.sync_copy(src_hbm.at[s], sidx_all)
    pltpu.sync_copy(dst_hbm.at[s], didx_all)
    pltpu.sync_copy(z_hbm, agg_sh.at[pl.ds(s * _ZROWS, _ZROWS)])
    plsc.subcore_barrier()
    for group in _GROUPS:
        gathers = []
        for j, ch in enumerate(group):
            gathers.append(pltpu.async_copy(
                x_hbm.at[sidx_all.at[ch]], rows_v.at[j], gsem))
        scatters = []
        for j, ch in enumerate(group):
            gathers[j].wait()
            for k in range(_CH // _L):
                d = didx_all[ch, pl.ds(k * _L, _L)]
                loc = d - base
                ok = (loc >= 0) & (loc < _HALF)
                lidx[j][pl.ds(k * _L, _L)] = jnp.where(ok, loc, trash)
            scatters.append(pltpu.async_copy(
                rows_v.at[j], agg_sh.at[lidx[j]], ssem, add=True))
        for sc in scatters:
            sc.wait()
    plsc.subcore_barrier()

    @pl.when(s < _HALF // _CROWS)
    def _copy_out():
        pltpu.sync_copy(agg_sh.at[pl.ds(s * _CROWS, _CROWS)],
                        out_hbm.at[pl.ds(c * _HALF + s * _CROWS, _CROWS)])


_sc_agg = functools.partial(
    pl.kernel,
    mesh=plsc.VectorSubcoreMesh(core_axis_name="c", subcore_axis_name="s"),
    out_type=jax.ShapeDtypeStruct((_N, _D), jnp.float32),
    scratch_types=[
        pltpu.VMEM((_NCH, _CH), jnp.int32),
        pltpu.VMEM((_NCH, _CH), jnp.int32),
        pltpu.VMEM((_DEPTH, _CH, _D), jnp.float32),
        pltpu.VMEM((_CH,), jnp.int32),
        pltpu.VMEM((_CH,), jnp.int32),
        pltpu.VMEM((_CH,), jnp.int32),
        pltpu.VMEM((_CH,), jnp.int32),
        pltpu.VMEM((_CH,), jnp.int32),
        pltpu.VMEM((_CH,), jnp.int32),
        pltpu.VMEM_SHARED((_NPAD, _D), jnp.float32),
        pltpu.SemaphoreType.DMA,
        pltpu.SemaphoreType.DMA,
    ],
)(_sc_agg_body)


def _dense_body(agg_ref, xd_ref, wr_ref, wt_ref, b_ref, o_ref):
    acc = jnp.dot(agg_ref[...], wr_ref[...], preferred_element_type=jnp.float32)
    acc = acc + jnp.dot(xd_ref[...], wt_ref[...], preferred_element_type=jnp.float32)
    acc = acc + b_ref[...]
    o_ref[...] = jnp.maximum(acc, 0.0)


_BLK = 1000


def _dense(agg, xd, wr, wt, b2):
    return pl.pallas_call(
        _dense_body,
        grid=(_N // _BLK,),
        in_specs=[
            pl.BlockSpec((_BLK, _D), lambda i: (i, 0)),
            pl.BlockSpec((_BLK, _D), lambda i: (i, 0)),
            pl.BlockSpec((_D, _D), lambda i: (0, 0)),
            pl.BlockSpec((_D, _D), lambda i: (0, 0)),
            pl.BlockSpec((1, _D), lambda i: (0, 0)),
        ],
        out_specs=pl.BlockSpec((_BLK, _D), lambda i: (i, 0)),
        out_shape=jax.ShapeDtypeStruct((_N, _D), jnp.float32),
    )(agg, xd, wr, wt, b2)


def kernel(x_router, x_pe, x_task, x_start_task, x_end_task, x_link, ei_router_to_router, ei_router_to_pe, ei_pe_to_router, ei_task_to_pe, ei_start_task_to_task, ei_task_to_task, ei_task_to_end_task, ei_task_to_link, ei_start_task_to_link, ei_link_to_router, ei_link_to_task, ei_link_to_end_task, ei_end_task_to_pe, ei_start_task_to_pe, W_rel_router_to_router, b_rel_router_to_router, W_root_router_to_router, W_rel_router_to_pe, b_rel_router_to_pe, W_root_router_to_pe, W_rel_pe_to_router, b_rel_pe_to_router, W_root_pe_to_router, W_rel_task_to_pe, b_rel_task_to_pe, W_root_task_to_pe, W_rel_start_task_to_task, b_rel_start_task_to_task, W_root_start_task_to_task, W_rel_task_to_task, b_rel_task_to_task, W_root_task_to_task, W_rel_task_to_end_task, b_rel_task_to_end_task, W_root_task_to_end_task, W_rel_task_to_link, b_rel_task_to_link, W_root_task_to_link, W_rel_start_task_to_link, b_rel_start_task_to_link, W_root_start_task_to_link, W_rel_link_to_router, b_rel_link_to_router, W_root_link_to_router, W_rel_link_to_task, b_rel_link_to_task, W_root_link_to_task, W_rel_link_to_end_task, b_rel_link_to_end_task, W_root_link_to_end_task, W_rel_end_task_to_pe, b_rel_end_task_to_pe, W_root_end_task_to_pe, W_rel_start_task_to_pe, b_rel_start_task_to_pe, W_root_start_task_to_pe):
    inp = dict(locals())
    x = {t: inp["x_" + t] for t in _TYPES}
    zeros = jnp.zeros((_ZROWS, _D), jnp.float32)

    prep = {}
    for (s, d) in _RELS:
        name = s + "_to_" + d
        ei = inp["ei_" + name]
        src = jnp.concatenate([ei[0], jnp.zeros((_EPAD - _E,), jnp.int32)])
        dst = jnp.concatenate([ei[1], jnp.full((_EPAD - _E,), _N, jnp.int32)])
        prep[name] = (src.reshape(_NS, _NCH, _CH), dst.reshape(_NS, _NCH, _CH))

    const_agg = {}
    for _ in range(2):
        for (s, d) in _RELS:
            name = s + "_to_" + d
            if s == "start_task":
                if name not in const_agg:
                    const_agg[name] = _sc_agg(x[s], *prep[name], zeros)
                agg = const_agg[name]
            else:
                agg = _sc_agg(x[s], *prep[name], zeros)
            x[d] = _dense(agg, x[d], inp["W_rel_" + name],
                          inp["W_root_" + name], inp["b_rel_" + name].reshape(1, _D))
    return tuple(x[t] for t in _TYPES)


# speedup vs baseline: 1.0060x; 1.0060x over previous
import functools

import jax
import jax.numpy as jnp
from jax import lax
from jax.experimental import pallas as pl
from jax.experimental.pallas import tpu as pltpu
from jax.experimental.pallas import tpu_sc as plsc

_N = 16000
_D = 128
_E = 40000

_NC = 2
_NS = 16
_HALF = _N // _NC
_L = 16

_CH = 64
_NCH = 40
_EPAD = _NS * _NCH * _CH
_NPAD = 8064
_ZROWS = _NPAD // _NS
_CROWS = 1000

_TYPES = ["router", "pe", "task", "start_task", "end_task", "link"]
_RELS = [("router", "router"), ("router", "pe"), ("pe", "router"), ("task", "pe"),
         ("start_task", "task"), ("task", "task"), ("task", "end_task"),
         ("task", "link"), ("start_task", "link"), ("link", "router"),
         ("link", "task"), ("link", "end_task"), ("end_task", "pe"),
         ("start_task", "pe")]


_DEPTH = 5


def _sc_agg_body(x_hbm, src_hbm, dst_hbm, z_hbm, out_hbm,
                 sidx_all, didx_all, rows_v,
                 lidx0, lidx1, lidx2, lidx3, lidx4,
                 agg_sh, gsem, ssem):
    lidx = (lidx0, lidx1, lidx2, lidx3, lidx4)
    c = lax.axis_index("c")
    s = lax.axis_index("s")
    base = c * _HALF
    trash = _HALF + s
    pltpu.sync_copy(src_hbm.at[s], sidx_all)
    pltpu.sync_copy(dst_hbm.at[s], didx_all)
    pltpu.sync_copy(z_hbm, agg_sh.at[pl.ds(s * _ZROWS, _ZROWS)])
    plsc.subcore_barrier()
    def _group(g, carry):
        gathers = []
        for j in range(_DEPTH):
            ch = g * _DEPTH + j
            gathers.append(pltpu.async_copy(
                x_hbm.at[sidx_all.at[ch]], rows_v.at[j], gsem))
        scatters = []
        for j in range(_DEPTH):
            ch = g * _DEPTH + j
            gathers[j].wait()
            for k in range(_CH // _L):
                d = didx_all[ch, pl.ds(k * _L, _L)]
                loc = d - base
                ok = (loc >= 0) & (loc < _HALF)
                lidx[j][pl.ds(k * _L, _L)] = jnp.where(ok, loc, trash)
            scatters.append(pltpu.async_copy(
                rows_v.at[j], agg_sh.at[lidx[j]], ssem, add=True))
        for sc in scatters:
            sc.wait()
        return carry

    lax.fori_loop(0, _NCH // _DEPTH, _group, 0)
    plsc.subcore_barrier()

    @pl.when(s < _HALF // _CROWS)
    def _copy_out():
        pltpu.sync_copy(agg_sh.at[pl.ds(s * _CROWS, _CROWS)],
                        out_hbm.at[pl.ds(c * _HALF + s * _CROWS, _CROWS)])


_sc_agg = functools.partial(
    pl.kernel,
    mesh=plsc.VectorSubcoreMesh(core_axis_name="c", subcore_axis_name="s"),
    out_type=jax.ShapeDtypeStruct((_N, _D), jnp.float32),
    scratch_types=[
        pltpu.VMEM((_NCH, _CH), jnp.int32),
        pltpu.VMEM((_NCH, _CH), jnp.int32),
        pltpu.VMEM((_DEPTH, _CH, _D), jnp.float32),
        pltpu.VMEM((_CH,), jnp.int32),
        pltpu.VMEM((_CH,), jnp.int32),
        pltpu.VMEM((_CH,), jnp.int32),
        pltpu.VMEM((_CH,), jnp.int32),
        pltpu.VMEM((_CH,), jnp.int32),
        pltpu.VMEM_SHARED((_NPAD, _D), jnp.float32),
        pltpu.SemaphoreType.DMA,
        pltpu.SemaphoreType.DMA,
    ],
)(_sc_agg_body)


def _dense_body(agg_ref, xd_ref, wr_ref, wt_ref, b_ref, o_ref):
    acc = jnp.dot(agg_ref[...], wr_ref[...], preferred_element_type=jnp.float32)
    acc = acc + jnp.dot(xd_ref[...], wt_ref[...], preferred_element_type=jnp.float32)
    acc = acc + b_ref[...]
    o_ref[...] = jnp.maximum(acc, 0.0)


_BLK = 1000


def _dense(agg, xd, wr, wt, b2):
    return pl.pallas_call(
        _dense_body,
        grid=(_N // _BLK,),
        in_specs=[
            pl.BlockSpec((_BLK, _D), lambda i: (i, 0)),
            pl.BlockSpec((_BLK, _D), lambda i: (i, 0)),
            pl.BlockSpec((_D, _D), lambda i: (0, 0)),
            pl.BlockSpec((_D, _D), lambda i: (0, 0)),
            pl.BlockSpec((1, _D), lambda i: (0, 0)),
        ],
        out_specs=pl.BlockSpec((_BLK, _D), lambda i: (i, 0)),
        out_shape=jax.ShapeDtypeStruct((_N, _D), jnp.float32),
    )(agg, xd, wr, wt, b2)


def kernel(x_router, x_pe, x_task, x_start_task, x_end_task, x_link, ei_router_to_router, ei_router_to_pe, ei_pe_to_router, ei_task_to_pe, ei_start_task_to_task, ei_task_to_task, ei_task_to_end_task, ei_task_to_link, ei_start_task_to_link, ei_link_to_router, ei_link_to_task, ei_link_to_end_task, ei_end_task_to_pe, ei_start_task_to_pe, W_rel_router_to_router, b_rel_router_to_router, W_root_router_to_router, W_rel_router_to_pe, b_rel_router_to_pe, W_root_router_to_pe, W_rel_pe_to_router, b_rel_pe_to_router, W_root_pe_to_router, W_rel_task_to_pe, b_rel_task_to_pe, W_root_task_to_pe, W_rel_start_task_to_task, b_rel_start_task_to_task, W_root_start_task_to_task, W_rel_task_to_task, b_rel_task_to_task, W_root_task_to_task, W_rel_task_to_end_task, b_rel_task_to_end_task, W_root_task_to_end_task, W_rel_task_to_link, b_rel_task_to_link, W_root_task_to_link, W_rel_start_task_to_link, b_rel_start_task_to_link, W_root_start_task_to_link, W_rel_link_to_router, b_rel_link_to_router, W_root_link_to_router, W_rel_link_to_task, b_rel_link_to_task, W_root_link_to_task, W_rel_link_to_end_task, b_rel_link_to_end_task, W_root_link_to_end_task, W_rel_end_task_to_pe, b_rel_end_task_to_pe, W_root_end_task_to_pe, W_rel_start_task_to_pe, b_rel_start_task_to_pe, W_root_start_task_to_pe):
    inp = dict(locals())
    x = {t: inp["x_" + t] for t in _TYPES}
    zeros = jnp.zeros((_ZROWS, _D), jnp.float32)

    prep = {}
    for (s, d) in _RELS:
        name = s + "_to_" + d
        ei = inp["ei_" + name]
        src = jnp.concatenate([ei[0], jnp.zeros((_EPAD - _E,), jnp.int32)])
        dst = jnp.concatenate([ei[1], jnp.full((_EPAD - _E,), _N, jnp.int32)])
        prep[name] = (src.reshape(_NS, _NCH, _CH), dst.reshape(_NS, _NCH, _CH))

    const_agg = {}
    for _ in range(2):
        for (s, d) in _RELS:
            name = s + "_to_" + d
            if s == "start_task":
                if name not in const_agg:
                    const_agg[name] = _sc_agg(x[s], *prep[name], zeros)
                agg = const_agg[name]
            else:
                agg = _sc_agg(x[s], *prep[name], zeros)
            x[d] = _dense(agg, x[d], inp["W_rel_" + name],
                          inp["W_root_" + name], inp["b_rel_" + name].reshape(1, _D))
    return tuple(x[t] for t in _TYPES)
